# baseline (speedup 1.0000x reference)
import functools
import math

import jax
import jax.numpy as jnp
from jax import lax
from jax.experimental import pallas as pl
from jax.experimental.pallas import tpu as pltpu
from jax.experimental.pallas import tpu_sc as plsc

E = 64
D = 128
FF = 256
OD = 128
CAPF = 1.5


def _gate_body(x_ref, emb_ref, wg_ref, xf_ref, gate_ref, eidx_ref):
    xr = x_ref[...] + emb_ref[...]
    xf_ref[...] = xr
    logits = jnp.dot(xr, wg_ref[...], preferred_element_type=jnp.float32)
    m = jnp.max(logits, axis=1, keepdims=True)
    ex = jnp.exp(logits - m)
    s = jnp.sum(ex, axis=1, keepdims=True)
    scores = ex / s
    gate_ref[...] = jnp.max(scores, axis=1)
    eidx_ref[...] = jnp.argmax(scores, axis=1).astype(jnp.int32)


def _lane_cumsum(m, width):
    k = 1
    while k < width:
        z = jnp.zeros_like(m[:, :k])
        m = m + jnp.concatenate([z, m[:, :-k]], axis=1)
        k *= 2
    return m


def _routing_body(cap, chunks, ch, gate_ref, eidx_ref, dest_ref, scale_ref,
                  thr_ref, r_ref, gm_ref, tflag_ref):
    iota_e = lax.broadcasted_iota(jnp.int32, (E, 1), 0)
    ecap = E * cap

    def chunk(i):
        g = gate_ref[i]
        gb = lax.bitcast_convert_type(g, jnp.int32)
        e = eidx_ref[i]
        eoh = e == iota_e
        return g, gb, e, eoh

    def pre_chunk(i, acc):
        _, gb, _, eoh = chunk(i)
        gm_ref[i] = jnp.where(eoh, gb, 0)
        return acc + jnp.sum(eoh.astype(jnp.int32), axis=1, keepdims=True)

    loads = lax.fori_loop(0, chunks, pre_chunk, jnp.zeros((E, 1), jnp.int32))
    thr_ref[...] = jnp.zeros((E, 1), jnp.int32)
    r_ref[...] = jnp.full((E, 1), cap, jnp.int32)
    tflag_ref[0] = 0

    @pl.when(jnp.max(loads) > cap)
    def _search():
        def bs_iter(_, lohi):
            lo, hi = lohi
            mid = lo + (hi - lo + 1) // 2

            def count_chunk(i, acc):
                c = jnp.sum((gm_ref[i] >= mid).astype(jnp.int32), axis=1,
                            keepdims=True)
                return acc + c

            cnt = lax.fori_loop(0, chunks, count_chunk,
                                jnp.zeros((E, 1), jnp.int32))
            ok = cnt >= cap
            return jnp.where(ok, mid, lo), jnp.where(ok, hi, mid - 1)

        lo0 = jnp.full((E, 1), 0x3C000000, jnp.int32)
        hi0 = jnp.full((E, 1), 0x3F800000, jnp.int32)
        thr, _ = lax.fori_loop(0, 26, bs_iter, (lo0, hi0))

        def gt_chunk(i, acc):
            return acc + jnp.sum((gm_ref[i] > thr).astype(jnp.int32), axis=1,
                                 keepdims=True)

        c_gt = lax.fori_loop(0, chunks, gt_chunk,
                             jnp.zeros((E, 1), jnp.int32))

        def ge_chunk(i, acc):
            return acc + jnp.sum((gm_ref[i] >= thr).astype(jnp.int32),
                                 axis=1, keepdims=True)

        n_ge = lax.fori_loop(0, chunks, ge_chunk,
                             jnp.zeros((E, 1), jnp.int32))
        thr_ref[...] = jnp.where(loads > cap, thr, 0)
        r_ref[...] = cap - jnp.where(loads > cap, c_gt, 0)
        tflag_ref[0] = jnp.any((loads > cap) & (n_ge > cap)).astype(jnp.int32)

    thr = thr_ref[...]
    r = r_ref[...]

    z = jnp.zeros((E, 1), jnp.int32)

    @pl.when(tflag_ref[0] == 1)
    def _scan_heavy():
        def scan_chunk(i, carry):
            kcnt, tcnt = carry
            g, gb, e, eoh = chunk(i)
            eoh_i = eoh.astype(jnp.int32)
            t_tok = jnp.sum(jnp.where(eoh, thr, 0), axis=0, keepdims=True)
            r_tok = jnp.sum(jnp.where(eoh, r, 0), axis=0, keepdims=True)
            gt = gb > t_tok
            tie = gb == t_tok
            tie_m = eoh_i * tie.astype(jnp.int32)
            tpre = _lane_cumsum(tie_m, ch) - tie_m
            trank = jnp.sum(jnp.where(eoh, tpre + tcnt, 0), axis=0,
                            keepdims=True)
            keep = gt | (tie & (trank < r_tok))
            keep_m = eoh_i * keep.astype(jnp.int32)
            kpre = _lane_cumsum(keep_m, ch) - keep_m
            slot = jnp.sum(jnp.where(eoh, kpre + kcnt, 0), axis=0,
                           keepdims=True)
            dest_ref[i] = jnp.where(keep, e * cap + slot, ecap)
            scale_ref[i] = jnp.where(keep, g, 0.0)
            kcnt = kcnt + jnp.sum(keep_m, axis=1, keepdims=True)
            tcnt = tcnt + jnp.sum(tie_m, axis=1, keepdims=True)
            return kcnt, tcnt

        lax.fori_loop(0, chunks, scan_chunk, (z, z))

    @pl.when(tflag_ref[0] == 0)
    def _scan_light():
        def scan_chunk(i, kcnt):
            g, gb, e, eoh = chunk(i)
            t_tok = jnp.sum(jnp.where(eoh, thr, 0), axis=0, keepdims=True)
            keep = gb >= t_tok
            keep_m = eoh.astype(jnp.int32) * keep.astype(jnp.int32)
            kpre = _lane_cumsum(keep_m, ch) - keep_m
            slot = jnp.sum(jnp.where(eoh, kpre + kcnt, 0), axis=0,
                           keepdims=True)
            dest_ref[i] = jnp.where(keep, e * cap + slot, ecap)
            scale_ref[i] = jnp.where(keep, g, 0.0)
            return kcnt + jnp.sum(keep_m, axis=1, keepdims=True)

        lax.fori_loop(0, chunks, scan_chunk, z)


def _ffn_body(nblk, buf_ref, sb_ref, w1_ref, b1_ref, w2_ref, b2_ref, wp_ref,
              bp_ref, y_ref):
    xb = buf_ref[...].astype(jnp.bfloat16)
    h = jnp.dot(xb, w1_ref[0].astype(jnp.bfloat16),
                preferred_element_type=jnp.float32)
    h = jax.nn.gelu(h + b1_ref[0])
    o = jnp.dot(h.astype(jnp.bfloat16), w2_ref[0].astype(jnp.bfloat16),
                preferred_element_type=jnp.float32)
    o = o + b2_ref[0]
    s = sb_ref[...][:, None]
    t = jax.nn.gelu(jnp.where(s > 0, o, 0.0) * s)
    y_ref[...] = (jnp.dot(t, wp_ref[...], preferred_element_type=jnp.float32)
                  + bp_ref[...])

    @pl.when(pl.program_id(0) == nblk - 1)
    def _():
        y_ref[...] = jnp.broadcast_to(bp_ref[...], y_ref.shape)


def _dispatch_body(n, xf_hbm, dest_hbm, scale_hbm, buf_hbm, sbuf_hbm,
                   idx_v, rows_v, sv_v, sem, sem2):
    wid = lax.axis_index("s") * 2 + lax.axis_index("c")
    per_w = n // 32
    base0 = wid * per_w

    def body(j, carry):
        base = base0 + j * 128
        pltpu.sync_copy(dest_hbm.at[pl.ds(base, 128)], idx_v)
        pltpu.sync_copy(xf_hbm.at[pl.ds(base, 128)], rows_v)
        pltpu.sync_copy(scale_hbm.at[pl.ds(base, 128)], sv_v)
        cp1 = pltpu.async_copy(rows_v, buf_hbm.at[idx_v], sem)
        cp2 = pltpu.async_copy(sv_v, sbuf_hbm.at[idx_v], sem2)
        cp1.wait()
        cp2.wait()
        return carry

    lax.fori_loop(0, per_w // 128, body, 0)


def _combine_body(n, o_hbm, dest_hbm, out_hbm, idx_v, rows_v, sem):
    wid = lax.axis_index("s") * 2 + lax.axis_index("c")
    per_w = n // 32
    base0 = wid * per_w

    def body(j, carry):
        base = base0 + j * 128
        pltpu.sync_copy(dest_hbm.at[pl.ds(base, 128)], idx_v)
        pltpu.async_copy(o_hbm.at[idx_v], rows_v, sem).wait()
        pltpu.sync_copy(rows_v, out_hbm.at[pl.ds(base, 128)])
        return carry

    lax.fori_loop(0, per_w // 128, body, 0)


def kernel(x, embedding, Wg, W1, b1, W2, b2, Wp, bp):
    B, T, H, _ = x.shape
    N = B * T * H
    TH = T * H
    cap = int(math.ceil(CAPF * N / E))
    ecap = E * cap

    BLKA = 2048
    CH = 8192
    CHUNKS = N // CH
    BLKD = 512
    RB = ecap + BLKD
    CPB = cap // BLKD
    BLKF = 2048

    x2 = x.reshape(N, D)
    emb2 = embedding.reshape(TH, D)

    xf, gate, eidx = pl.pallas_call(
        _gate_body,
        grid=(N // BLKA,),
        in_specs=[
            pl.BlockSpec((BLKA, D), lambda i: (i, 0)),
            pl.BlockSpec((BLKA, D), lambda i: (i % (TH // BLKA), 0)),
            pl.BlockSpec((D, E), lambda i: (0, 0)),
        ],
        out_specs=[
            pl.BlockSpec((BLKA, D), lambda i: (i, 0)),
            pl.BlockSpec((BLKA,), lambda i: (i,)),
            pl.BlockSpec((BLKA,), lambda i: (i,)),
        ],
        out_shape=[
            jax.ShapeDtypeStruct((N, D), jnp.float32),
            jax.ShapeDtypeStruct((N,), jnp.float32),
            jax.ShapeDtypeStruct((N,), jnp.int32),
        ],
    )(x2, emb2, Wg)

    g3 = gate.reshape(CHUNKS, 1, CH)
    e3 = eidx.reshape(CHUNKS, 1, CH)
    dest3, scale3 = pl.pallas_call(
        functools.partial(_routing_body, cap, CHUNKS, CH),
        out_shape=[
            jax.ShapeDtypeStruct((CHUNKS, 1, CH), jnp.int32),
            jax.ShapeDtypeStruct((CHUNKS, 1, CH), jnp.float32),
        ],
        scratch_shapes=[
            pltpu.VMEM((E, 1), jnp.int32),
            pltpu.VMEM((E, 1), jnp.int32),
            pltpu.VMEM((CHUNKS, E, CH), jnp.int32),
            pltpu.SMEM((1,), jnp.int32),
        ],
    )(g3, e3)
    dest = dest3.reshape(N)
    scale = scale3.reshape(N)

    mesh = plsc.VectorSubcoreMesh(core_axis_name="c", subcore_axis_name="s")
    buf, sbuf = pl.kernel(
        functools.partial(_dispatch_body, N),
        mesh=mesh,
        out_type=(
            jax.ShapeDtypeStruct((RB, D), jnp.float32),
            jax.ShapeDtypeStruct((RB,), jnp.float32),
        ),
        scratch_types=[
            pltpu.VMEM((128,), jnp.int32),
            pltpu.VMEM((128, D), jnp.float32),
            pltpu.VMEM((128,), jnp.float32),
            pltpu.SemaphoreType.DMA,
            pltpu.SemaphoreType.DMA,
        ],
    )(xf, dest, scale)

    NBLK = RB // BLKD
    yslot = pl.pallas_call(
        functools.partial(_ffn_body, NBLK),
        grid=(NBLK,),
        in_specs=[
            pl.BlockSpec((BLKD, D), lambda i: (i, 0)),
            pl.BlockSpec((BLKD,), lambda i: (i,)),
            pl.BlockSpec((1, D, FF),
                         lambda i: (jnp.minimum(i // CPB, E - 1), 0, 0)),
            pl.BlockSpec((1, 1, FF),
                         lambda i: (jnp.minimum(i // CPB, E - 1), 0, 0)),
            pl.BlockSpec((1, FF, D),
                         lambda i: (jnp.minimum(i // CPB, E - 1), 0, 0)),
            pl.BlockSpec((1, 1, D),
                         lambda i: (jnp.minimum(i // CPB, E - 1), 0, 0)),
            pl.BlockSpec((D, OD), lambda i: (0, 0)),
            pl.BlockSpec((OD,), lambda i: (0,)),
        ],
        out_specs=pl.BlockSpec((BLKD, OD), lambda i: (i, 0)),
        out_shape=jax.ShapeDtypeStruct((RB, OD), jnp.float32),
    )(buf, sbuf, W1, b1.reshape(E, 1, FF), W2, b2.reshape(E, 1, D), Wp, bp)

    y = pl.kernel(
        functools.partial(_combine_body, N),
        mesh=mesh,
        out_type=jax.ShapeDtypeStruct((N, OD), jnp.float32),
        scratch_types=[
            pltpu.VMEM((128,), jnp.int32),
            pltpu.VMEM((128, OD), jnp.float32),
            pltpu.SemaphoreType.DMA,
        ],
    )(yslot, dest)

    return y.reshape(B, T, H, OD)


# speedup vs baseline: 1.2268x; 1.2268x over previous
import functools
import math

import jax
import jax.numpy as jnp
from jax import lax
from jax.experimental import pallas as pl
from jax.experimental.pallas import tpu as pltpu
from jax.experimental.pallas import tpu_sc as plsc

E = 64
D = 128
FF = 256
OD = 128
CAPF = 1.5


def _gate_body(x_ref, emb_ref, wg_ref, xf_ref, gate_ref, eidx_ref):
    xr = x_ref[...] + emb_ref[...]
    xf_ref[...] = xr
    logits = jnp.dot(xr, wg_ref[...], preferred_element_type=jnp.float32)
    m = jnp.max(logits, axis=1, keepdims=True)
    ex = jnp.exp(logits - m)
    s = jnp.sum(ex, axis=1, keepdims=True)
    scores = ex / s
    gate_ref[...] = jnp.max(scores, axis=1)
    eidx_ref[...] = jnp.argmax(scores, axis=1).astype(jnp.int32)


def _lane_cumsum(m, width):
    k = 1
    while k < width:
        z = jnp.zeros_like(m[:, :k])
        m = m + jnp.concatenate([z, m[:, :-k]], axis=1)
        k *= 2
    return m


def _routing_body(cap, chunks, ch, gate_ref, eidx_ref, dest_ref, scale_ref,
                  thr_ref, r_ref, gm_ref, tflag_ref):
    iota_e = lax.broadcasted_iota(jnp.int32, (E, 1), 0)
    ecap = E * cap

    def chunk(i):
        g = gate_ref[i]
        gb = lax.bitcast_convert_type(g, jnp.int32)
        e = eidx_ref[i]
        eoh = e == iota_e
        return g, gb, e, eoh

    def pre_chunk(i, acc):
        _, gb, _, eoh = chunk(i)
        gm_ref[i] = jnp.where(eoh, gb, 0)
        return acc + jnp.sum(eoh.astype(jnp.int32), axis=1, keepdims=True)

    loads = lax.fori_loop(0, chunks, pre_chunk, jnp.zeros((E, 1), jnp.int32))
    thr_ref[...] = jnp.zeros((E, 1), jnp.int32)
    r_ref[...] = jnp.full((E, 1), cap, jnp.int32)
    tflag_ref[0] = 0

    @pl.when(jnp.max(loads) > cap)
    def _search():
        def bs_iter(_, lohi):
            lo, hi = lohi
            mid = lo + (hi - lo + 1) // 2

            def count_chunk(i, acc):
                c = jnp.sum((gm_ref[i] >= mid).astype(jnp.int32), axis=1,
                            keepdims=True)
                return acc + c

            cnt = lax.fori_loop(0, chunks, count_chunk,
                                jnp.zeros((E, 1), jnp.int32))
            ok = cnt >= cap
            return jnp.where(ok, mid, lo), jnp.where(ok, hi, mid - 1)

        lo0 = jnp.full((E, 1), 0x3C000000, jnp.int32)
        hi0 = jnp.full((E, 1), 0x3F800000, jnp.int32)
        thr, _ = lax.fori_loop(0, 26, bs_iter, (lo0, hi0))

        def gt_chunk(i, acc):
            return acc + jnp.sum((gm_ref[i] > thr).astype(jnp.int32), axis=1,
                                 keepdims=True)

        c_gt = lax.fori_loop(0, chunks, gt_chunk,
                             jnp.zeros((E, 1), jnp.int32))

        def ge_chunk(i, acc):
            return acc + jnp.sum((gm_ref[i] >= thr).astype(jnp.int32),
                                 axis=1, keepdims=True)

        n_ge = lax.fori_loop(0, chunks, ge_chunk,
                             jnp.zeros((E, 1), jnp.int32))
        thr_ref[...] = jnp.where(loads > cap, thr, 0)
        r_ref[...] = cap - jnp.where(loads > cap, c_gt, 0)
        tflag_ref[0] = jnp.any((loads > cap) & (n_ge > cap)).astype(jnp.int32)

    thr = thr_ref[...]
    r = r_ref[...]

    z = jnp.zeros((E, 1), jnp.int32)

    @pl.when(tflag_ref[0] == 1)
    def _scan_heavy():
        def scan_chunk(i, carry):
            kcnt, tcnt = carry
            g, gb, e, eoh = chunk(i)
            eoh_i = eoh.astype(jnp.int32)
            t_tok = jnp.sum(jnp.where(eoh, thr, 0), axis=0, keepdims=True)
            r_tok = jnp.sum(jnp.where(eoh, r, 0), axis=0, keepdims=True)
            gt = gb > t_tok
            tie = gb == t_tok
            tie_m = eoh_i * tie.astype(jnp.int32)
            tpre = _lane_cumsum(tie_m, ch) - tie_m
            trank = jnp.sum(jnp.where(eoh, tpre + tcnt, 0), axis=0,
                            keepdims=True)
            keep = gt | (tie & (trank < r_tok))
            keep_m = eoh_i * keep.astype(jnp.int32)
            kpre = _lane_cumsum(keep_m, ch) - keep_m
            slot = jnp.sum(jnp.where(eoh, kpre + kcnt, 0), axis=0,
                           keepdims=True)
            dest_ref[i] = jnp.where(keep, e * cap + slot, ecap)
            scale_ref[i] = jnp.where(keep, g, 0.0)
            kcnt = kcnt + jnp.sum(keep_m, axis=1, keepdims=True)
            tcnt = tcnt + jnp.sum(tie_m, axis=1, keepdims=True)
            return kcnt, tcnt

        lax.fori_loop(0, chunks, scan_chunk, (z, z))

    @pl.when(tflag_ref[0] == 0)
    def _scan_light():
        def scan_chunk(i, kcnt):
            g, gb, e, eoh = chunk(i)
            t_tok = jnp.sum(jnp.where(eoh, thr, 0), axis=0, keepdims=True)
            keep = gb >= t_tok
            keep_m = eoh.astype(jnp.int32) * keep.astype(jnp.int32)
            kpre = _lane_cumsum(keep_m, ch) - keep_m
            slot = jnp.sum(jnp.where(eoh, kpre + kcnt, 0), axis=0,
                           keepdims=True)
            dest_ref[i] = jnp.where(keep, e * cap + slot, ecap)
            scale_ref[i] = jnp.where(keep, g, 0.0)
            return kcnt + jnp.sum(keep_m, axis=1, keepdims=True)

        lax.fori_loop(0, chunks, scan_chunk, z)


def _ffn_body(buf_ref, w1_ref, b1_ref, w2_ref, b2_ref, o_ref):
    xb = buf_ref[...].astype(jnp.bfloat16)
    h = jnp.dot(xb, w1_ref[0].astype(jnp.bfloat16),
                preferred_element_type=jnp.float32)
    h = jax.nn.gelu(h + b1_ref[0])
    o = jnp.dot(h.astype(jnp.bfloat16), w2_ref[0].astype(jnp.bfloat16),
                preferred_element_type=jnp.float32)
    o_ref[...] = o + b2_ref[0]


def _out_body(ot_ref, scale_ref, wp_ref, bp_ref, y_ref):
    s = scale_ref[...][:, None]
    t = jnp.where(s > 0, ot_ref[...], 0.0) * s
    t = jax.nn.gelu(t)
    y_ref[...] = (jnp.dot(t, wp_ref[...], preferred_element_type=jnp.float32)
                  + bp_ref[...])


def _dispatch_body(n, xf_hbm, dest_hbm, buf_hbm, idx_v, rows_v, sem):
    wid = lax.axis_index("s") * 2 + lax.axis_index("c")
    per_w = n // 32
    base0 = wid * per_w

    def body(j, carry):
        base = base0 + j * 128
        pltpu.sync_copy(dest_hbm.at[pl.ds(base, 128)], idx_v)
        pltpu.sync_copy(xf_hbm.at[pl.ds(base, 128)], rows_v)
        pltpu.async_copy(rows_v, buf_hbm.at[idx_v], sem).wait()
        return carry

    lax.fori_loop(0, per_w // 128, body, 0)


def _combine_body(n, o_hbm, dest_hbm, out_hbm, idx_v, rows_v, sem):
    wid = lax.axis_index("s") * 2 + lax.axis_index("c")
    per_w = n // 32
    base0 = wid * per_w

    def body(j, carry):
        base = base0 + j * 128
        pltpu.sync_copy(dest_hbm.at[pl.ds(base, 128)], idx_v)
        pltpu.async_copy(o_hbm.at[idx_v], rows_v, sem).wait()
        pltpu.sync_copy(rows_v, out_hbm.at[pl.ds(base, 128)])
        return carry

    lax.fori_loop(0, per_w // 128, body, 0)


def kernel(x, embedding, Wg, W1, b1, W2, b2, Wp, bp):
    B, T, H, _ = x.shape
    N = B * T * H
    TH = T * H
    cap = int(math.ceil(CAPF * N / E))
    ecap = E * cap

    BLKA = 2048
    CH = 8192
    CHUNKS = N // CH
    BLKD = 1024
    RB = ecap + BLKD
    CPB = cap // BLKD
    BLKF = 2048

    x2 = x.reshape(N, D)
    emb2 = embedding.reshape(TH, D)

    xf, gate, eidx = pl.pallas_call(
        _gate_body,
        grid=(N // BLKA,),
        in_specs=[
            pl.BlockSpec((BLKA, D), lambda i: (i, 0)),
            pl.BlockSpec((BLKA, D), lambda i: (i % (TH // BLKA), 0)),
            pl.BlockSpec((D, E), lambda i: (0, 0)),
        ],
        out_specs=[
            pl.BlockSpec((BLKA, D), lambda i: (i, 0)),
            pl.BlockSpec((BLKA,), lambda i: (i,)),
            pl.BlockSpec((BLKA,), lambda i: (i,)),
        ],
        out_shape=[
            jax.ShapeDtypeStruct((N, D), jnp.float32),
            jax.ShapeDtypeStruct((N,), jnp.float32),
            jax.ShapeDtypeStruct((N,), jnp.int32),
        ],
    )(x2, emb2, Wg)

    g3 = gate.reshape(CHUNKS, 1, CH)
    e3 = eidx.reshape(CHUNKS, 1, CH)
    dest3, scale3 = pl.pallas_call(
        functools.partial(_routing_body, cap, CHUNKS, CH),
        out_shape=[
            jax.ShapeDtypeStruct((CHUNKS, 1, CH), jnp.int32),
            jax.ShapeDtypeStruct((CHUNKS, 1, CH), jnp.float32),
        ],
        scratch_shapes=[
            pltpu.VMEM((E, 1), jnp.int32),
            pltpu.VMEM((E, 1), jnp.int32),
            pltpu.VMEM((CHUNKS, E, CH), jnp.int32),
            pltpu.SMEM((1,), jnp.int32),
        ],
    )(g3, e3)
    dest = dest3.reshape(N)
    scale = scale3.reshape(N)

    mesh = plsc.VectorSubcoreMesh(core_axis_name="c", subcore_axis_name="s")
    buf = pl.kernel(
        functools.partial(_dispatch_body, N),
        mesh=mesh,
        out_type=jax.ShapeDtypeStruct((RB, D), jnp.float32),
        scratch_types=[
            pltpu.VMEM((128,), jnp.int32),
            pltpu.VMEM((128, D), jnp.float32),
            pltpu.SemaphoreType.DMA,
        ],
    )(xf, dest)

    o = pl.pallas_call(
        _ffn_body,
        grid=(RB // BLKD,),
        in_specs=[
            pl.BlockSpec((BLKD, D), lambda i: (i, 0)),
            pl.BlockSpec((1, D, FF),
                         lambda i: (jnp.minimum(i // CPB, E - 1), 0, 0)),
            pl.BlockSpec((1, 1, FF),
                         lambda i: (jnp.minimum(i // CPB, E - 1), 0, 0)),
            pl.BlockSpec((1, FF, D),
                         lambda i: (jnp.minimum(i // CPB, E - 1), 0, 0)),
            pl.BlockSpec((1, 1, D),
                         lambda i: (jnp.minimum(i // CPB, E - 1), 0, 0)),
        ],
        out_specs=pl.BlockSpec((BLKD, D), lambda i: (i, 0)),
        out_shape=jax.ShapeDtypeStruct((RB, D), jnp.float32),
    )(buf, W1, b1.reshape(E, 1, FF), W2, b2.reshape(E, 1, D))

    otok = pl.kernel(
        functools.partial(_combine_body, N),
        mesh=mesh,
        out_type=jax.ShapeDtypeStruct((N, D), jnp.float32),
        scratch_types=[
            pltpu.VMEM((128,), jnp.int32),
            pltpu.VMEM((128, D), jnp.float32),
            pltpu.SemaphoreType.DMA,
        ],
    )(o, dest)

    y = pl.pallas_call(
        _out_body,
        grid=(N // BLKF,),
        in_specs=[
            pl.BlockSpec((BLKF, D), lambda i: (i, 0)),
            pl.BlockSpec((BLKF,), lambda i: (i,)),
            pl.BlockSpec((D, OD), lambda i: (0, 0)),
            pl.BlockSpec((OD,), lambda i: (0,)),
        ],
        out_specs=pl.BlockSpec((BLKF, OD), lambda i: (i, 0)),
        out_shape=jax.ShapeDtypeStruct((N, OD), jnp.float32),
    )(otok, scale, Wp, bp)

    return y.reshape(B, T, H, OD)


# speedup vs baseline: 1.3650x; 1.1127x over previous
import functools
import math

import jax
import jax.numpy as jnp
from jax import lax
from jax.experimental import pallas as pl
from jax.experimental.pallas import tpu as pltpu
from jax.experimental.pallas import tpu_sc as plsc

E = 64
D = 128
FF = 256
OD = 128
CAPF = 1.5


def _gate_body(x_ref, emb_ref, wg_ref, xf_ref, gate_ref, eidx_ref):
    xr = x_ref[...] + emb_ref[...]
    xf_ref[...] = xr
    logits = jnp.dot(xr, wg_ref[...], preferred_element_type=jnp.float32)
    m = jnp.max(logits, axis=1, keepdims=True)
    ex = jnp.exp(logits - m)
    s = jnp.sum(ex, axis=1, keepdims=True)
    scores = ex / s
    gate_ref[...] = jnp.max(scores, axis=1)
    eidx_ref[...] = jnp.argmax(scores, axis=1).astype(jnp.int32)


def _lane_cumsum(m, width):
    k = 1
    while k < width:
        z = jnp.zeros_like(m[:, :k])
        m = m + jnp.concatenate([z, m[:, :-k]], axis=1)
        k *= 2
    return m


def _routing_body(cap, chunks, ch, gate_ref, eidx_ref, dest_ref, scale_ref,
                  thr_ref, r_ref, gm_ref, tflag_ref):
    iota_e = lax.broadcasted_iota(jnp.int32, (E, 1), 0)
    ecap = E * cap

    def chunk(i):
        g = gate_ref[i]
        gb = lax.bitcast_convert_type(g, jnp.int32)
        e = eidx_ref[i]
        eoh = e == iota_e
        return g, gb, e, eoh

    def pre_chunk(i, acc):
        _, gb, _, eoh = chunk(i)
        gm_ref[i] = jnp.where(eoh, gb, 0)
        return acc + jnp.sum(eoh.astype(jnp.int32), axis=1, keepdims=True)

    loads = lax.fori_loop(0, chunks, pre_chunk, jnp.zeros((E, 1), jnp.int32))
    thr_ref[...] = jnp.zeros((E, 1), jnp.int32)
    r_ref[...] = jnp.full((E, 1), cap, jnp.int32)
    tflag_ref[0] = 0

    @pl.when(jnp.max(loads) > cap)
    def _search():
        def bs_iter(_, lohi):
            lo, hi = lohi
            mid = lo + (hi - lo + 1) // 2

            def count_chunk(i, acc):
                c = jnp.sum((gm_ref[i] >= mid).astype(jnp.int32), axis=1,
                            keepdims=True)
                return acc + c

            cnt = lax.fori_loop(0, chunks, count_chunk,
                                jnp.zeros((E, 1), jnp.int32))
            ok = cnt >= cap
            return jnp.where(ok, mid, lo), jnp.where(ok, hi, mid - 1)

        lo0 = jnp.full((E, 1), 0x3C000000, jnp.int32)
        hi0 = jnp.full((E, 1), 0x3F800000, jnp.int32)
        thr, _ = lax.fori_loop(0, 26, bs_iter, (lo0, hi0))

        def gt_chunk(i, acc):
            return acc + jnp.sum((gm_ref[i] > thr).astype(jnp.int32), axis=1,
                                 keepdims=True)

        c_gt = lax.fori_loop(0, chunks, gt_chunk,
                             jnp.zeros((E, 1), jnp.int32))

        def ge_chunk(i, acc):
            return acc + jnp.sum((gm_ref[i] >= thr).astype(jnp.int32),
                                 axis=1, keepdims=True)

        n_ge = lax.fori_loop(0, chunks, ge_chunk,
                             jnp.zeros((E, 1), jnp.int32))
        thr_ref[...] = jnp.where(loads > cap, thr, 0)
        r_ref[...] = cap - jnp.where(loads > cap, c_gt, 0)
        tflag_ref[0] = jnp.any((loads > cap) & (n_ge > cap)).astype(jnp.int32)

    thr = thr_ref[...]
    r = r_ref[...]

    z = jnp.zeros((E, 1), jnp.int32)

    @pl.when(tflag_ref[0] == 1)
    def _scan_heavy():
        def scan_chunk(i, carry):
            kcnt, tcnt = carry
            g, gb, e, eoh = chunk(i)
            eoh_i = eoh.astype(jnp.int32)
            t_tok = jnp.sum(jnp.where(eoh, thr, 0), axis=0, keepdims=True)
            r_tok = jnp.sum(jnp.where(eoh, r, 0), axis=0, keepdims=True)
            gt = gb > t_tok
            tie = gb == t_tok
            tie_m = eoh_i * tie.astype(jnp.int32)
            tpre = _lane_cumsum(tie_m, ch) - tie_m
            trank = jnp.sum(jnp.where(eoh, tpre + tcnt, 0), axis=0,
                            keepdims=True)
            keep = gt | (tie & (trank < r_tok))
            keep_m = eoh_i * keep.astype(jnp.int32)
            kpre = _lane_cumsum(keep_m, ch) - keep_m
            slot = jnp.sum(jnp.where(eoh, kpre + kcnt, 0), axis=0,
                           keepdims=True)
            dest_ref[i] = jnp.where(keep, e * cap + slot, ecap)
            scale_ref[i] = jnp.where(keep, g, 0.0)
            kcnt = kcnt + jnp.sum(keep_m, axis=1, keepdims=True)
            tcnt = tcnt + jnp.sum(tie_m, axis=1, keepdims=True)
            return kcnt, tcnt

        lax.fori_loop(0, chunks, scan_chunk, (z, z))

    @pl.when(tflag_ref[0] == 0)
    def _scan_light():
        def scan_chunk(i, kcnt):
            g, gb, e, eoh = chunk(i)
            t_tok = jnp.sum(jnp.where(eoh, thr, 0), axis=0, keepdims=True)
            keep = gb >= t_tok
            keep_m = eoh.astype(jnp.int32) * keep.astype(jnp.int32)
            kpre = _lane_cumsum(keep_m, ch) - keep_m
            slot = jnp.sum(jnp.where(eoh, kpre + kcnt, 0), axis=0,
                           keepdims=True)
            dest_ref[i] = jnp.where(keep, e * cap + slot, ecap)
            scale_ref[i] = jnp.where(keep, g, 0.0)
            return kcnt + jnp.sum(keep_m, axis=1, keepdims=True)

        lax.fori_loop(0, chunks, scan_chunk, z)


def _ffn_body(buf_ref, w1_ref, b1_ref, w2_ref, b2_ref, o_ref):
    xb = buf_ref[...].astype(jnp.bfloat16)
    h = jnp.dot(xb, w1_ref[0].astype(jnp.bfloat16),
                preferred_element_type=jnp.float32)
    h = jax.nn.gelu(h + b1_ref[0])
    o = jnp.dot(h.astype(jnp.bfloat16), w2_ref[0].astype(jnp.bfloat16),
                preferred_element_type=jnp.float32)
    o_ref[...] = o + b2_ref[0]


def _out_body(ot_ref, scale_ref, wp_ref, bp_ref, y_ref):
    s = scale_ref[...][:, None]
    t = jnp.where(s > 0, ot_ref[...], 0.0) * s
    t = jax.nn.gelu(t)
    y_ref[...] = (jnp.dot(t, wp_ref[...], preferred_element_type=jnp.float32)
                  + bp_ref[...])


def _dispatch_body(n, xf_hbm, dest_hbm, buf_hbm, idx_v, rows_v, sem):
    wid = lax.axis_index("s") * 2 + lax.axis_index("c")
    per_w = n // 32
    base0 = wid * per_w

    def body(j, carry):
        base = base0 + j * 128
        pltpu.sync_copy(dest_hbm.at[pl.ds(base, 128)], idx_v)
        pltpu.sync_copy(xf_hbm.at[pl.ds(base, 128)], rows_v)
        pltpu.async_copy(rows_v, buf_hbm.at[idx_v], sem).wait()
        return carry

    lax.fori_loop(0, per_w // 128, body, 0)


def _combine_body(n, o_hbm, dest_hbm, out_hbm, idx_v, rows_v, sem):
    wid = lax.axis_index("s") * 2 + lax.axis_index("c")
    per_w = n // 32
    base0 = wid * per_w

    def body(j, carry):
        base = base0 + j * 128
        pltpu.sync_copy(dest_hbm.at[pl.ds(base, 128)], idx_v)
        pltpu.async_copy(o_hbm.at[idx_v], rows_v, sem).wait()
        pltpu.sync_copy(rows_v, out_hbm.at[pl.ds(base, 128)])
        return carry

    lax.fori_loop(0, per_w // 128, body, 0)


def kernel(x, embedding, Wg, W1, b1, W2, b2, Wp, bp):
    B, T, H, _ = x.shape
    N = B * T * H
    TH = T * H
    cap = int(math.ceil(CAPF * N / E))
    ecap = E * cap

    BLKA = 4096
    CH = 8192
    CHUNKS = N // CH
    BLKD = 3072
    RB = ecap + BLKD
    CPB = cap // BLKD
    BLKF = 2048

    x2 = x.reshape(N, D)
    emb2 = embedding.reshape(TH, D)

    xf, gate, eidx = pl.pallas_call(
        _gate_body,
        grid=(N // BLKA,),
        in_specs=[
            pl.BlockSpec((BLKA, D), lambda i: (i, 0)),
            pl.BlockSpec((BLKA, D), lambda i: (i % (TH // BLKA), 0)),
            pl.BlockSpec((D, E), lambda i: (0, 0)),
        ],
        out_specs=[
            pl.BlockSpec((BLKA, D), lambda i: (i, 0)),
            pl.BlockSpec((BLKA,), lambda i: (i,)),
            pl.BlockSpec((BLKA,), lambda i: (i,)),
        ],
        out_shape=[
            jax.ShapeDtypeStruct((N, D), jnp.float32),
            jax.ShapeDtypeStruct((N,), jnp.float32),
            jax.ShapeDtypeStruct((N,), jnp.int32),
        ],
    )(x2, emb2, Wg)

    g3 = gate.reshape(CHUNKS, 1, CH)
    e3 = eidx.reshape(CHUNKS, 1, CH)
    dest3, scale3 = pl.pallas_call(
        functools.partial(_routing_body, cap, CHUNKS, CH),
        out_shape=[
            jax.ShapeDtypeStruct((CHUNKS, 1, CH), jnp.int32),
            jax.ShapeDtypeStruct((CHUNKS, 1, CH), jnp.float32),
        ],
        scratch_shapes=[
            pltpu.VMEM((E, 1), jnp.int32),
            pltpu.VMEM((E, 1), jnp.int32),
            pltpu.VMEM((CHUNKS, E, CH), jnp.int32),
            pltpu.SMEM((1,), jnp.int32),
        ],
    )(g3, e3)
    dest = dest3.reshape(N)
    scale = scale3.reshape(N)

    mesh = plsc.VectorSubcoreMesh(core_axis_name="c", subcore_axis_name="s")
    buf = pl.kernel(
        functools.partial(_dispatch_body, N),
        mesh=mesh,
        out_type=jax.ShapeDtypeStruct((RB, D), jnp.float32),
        scratch_types=[
            pltpu.VMEM((128,), jnp.int32),
            pltpu.VMEM((128, D), jnp.float32),
            pltpu.SemaphoreType.DMA,
        ],
    )(xf, dest)

    o = pl.pallas_call(
        _ffn_body,
        grid=(RB // BLKD,),
        in_specs=[
            pl.BlockSpec((BLKD, D), lambda i: (i, 0)),
            pl.BlockSpec((1, D, FF),
                         lambda i: (jnp.minimum(i // CPB, E - 1), 0, 0)),
            pl.BlockSpec((1, 1, FF),
                         lambda i: (jnp.minimum(i // CPB, E - 1), 0, 0)),
            pl.BlockSpec((1, FF, D),
                         lambda i: (jnp.minimum(i // CPB, E - 1), 0, 0)),
            pl.BlockSpec((1, 1, D),
                         lambda i: (jnp.minimum(i // CPB, E - 1), 0, 0)),
        ],
        out_specs=pl.BlockSpec((BLKD, D), lambda i: (i, 0)),
        out_shape=jax.ShapeDtypeStruct((RB, D), jnp.float32),
    )(buf, W1, b1.reshape(E, 1, FF), W2, b2.reshape(E, 1, D))

    otok = pl.kernel(
        functools.partial(_combine_body, N),
        mesh=mesh,
        out_type=jax.ShapeDtypeStruct((N, D), jnp.float32),
        scratch_types=[
            pltpu.VMEM((128,), jnp.int32),
            pltpu.VMEM((128, D), jnp.float32),
            pltpu.SemaphoreType.DMA,
        ],
    )(o, dest)

    y = pl.pallas_call(
        _out_body,
        grid=(N // BLKF,),
        in_specs=[
            pl.BlockSpec((BLKF, D), lambda i: (i, 0)),
            pl.BlockSpec((BLKF,), lambda i: (i,)),
            pl.BlockSpec((D, OD), lambda i: (0, 0)),
            pl.BlockSpec((OD,), lambda i: (0,)),
        ],
        out_specs=pl.BlockSpec((BLKF, OD), lambda i: (i, 0)),
        out_shape=jax.ShapeDtypeStruct((N, OD), jnp.float32),
    )(otok, scale, Wp, bp)

    return y.reshape(B, T, H, OD)


# speedup vs baseline: 1.4145x; 1.0363x over previous
import functools
import math

import jax
import jax.numpy as jnp
from jax import lax
from jax.experimental import pallas as pl
from jax.experimental.pallas import tpu as pltpu
from jax.experimental.pallas import tpu_sc as plsc

E = 64
D = 128
FF = 256
OD = 128
CAPF = 1.5


def _gate_body(x_ref, emb_ref, wg_ref, xf_ref, gate_ref, eidx_ref):
    xr = x_ref[...] + emb_ref[...]
    xf_ref[...] = xr
    logits = jnp.dot(xr, wg_ref[...], preferred_element_type=jnp.float32)
    m = jnp.max(logits, axis=1, keepdims=True)
    ex = jnp.exp(logits - m)
    s = jnp.sum(ex, axis=1, keepdims=True)
    scores = ex / s
    gate_ref[...] = jnp.max(scores, axis=1)
    eidx_ref[...] = jnp.argmax(scores, axis=1).astype(jnp.int32)


def _lane_cumsum(m, width):
    k = 1
    while k < width:
        z = jnp.zeros_like(m[:, :k])
        m = m + jnp.concatenate([z, m[:, :-k]], axis=1)
        k *= 2
    return m


def _routing_body(cap, chunks, ch, gate_ref, eidx_ref, dest_ref, scale_ref,
                  thr_ref, r_ref, gm_ref, tflag_ref):
    iota_e = lax.broadcasted_iota(jnp.int32, (E, 1), 0)
    ecap = E * cap

    def chunk(i):
        g = gate_ref[i]
        gb = lax.bitcast_convert_type(g, jnp.int32)
        e = eidx_ref[i]
        eoh = e == iota_e
        return g, gb, e, eoh

    def pre_chunk(i, acc):
        _, gb, _, eoh = chunk(i)
        gm_ref[i] = jnp.where(eoh, gb, 0)
        return acc + jnp.sum(eoh.astype(jnp.int32), axis=1, keepdims=True)

    loads = lax.fori_loop(0, chunks, pre_chunk, jnp.zeros((E, 1), jnp.int32))
    thr_ref[...] = jnp.zeros((E, 1), jnp.int32)
    r_ref[...] = jnp.full((E, 1), cap, jnp.int32)
    tflag_ref[0] = 0

    @pl.when(jnp.max(loads) > cap)
    def _search():
        def bs_iter(_, lohi):
            lo, hi = lohi
            mid = lo + (hi - lo + 1) // 2

            def count_chunk(i, acc):
                c = jnp.sum((gm_ref[i] >= mid).astype(jnp.int32), axis=1,
                            keepdims=True)
                return acc + c

            cnt = lax.fori_loop(0, chunks, count_chunk,
                                jnp.zeros((E, 1), jnp.int32))
            ok = cnt >= cap
            return jnp.where(ok, mid, lo), jnp.where(ok, hi, mid - 1)

        lo0 = jnp.full((E, 1), 0x3C000000, jnp.int32)
        hi0 = jnp.full((E, 1), 0x3F800000, jnp.int32)
        thr, _ = lax.fori_loop(0, 26, bs_iter, (lo0, hi0))

        def gt_chunk(i, acc):
            return acc + jnp.sum((gm_ref[i] > thr).astype(jnp.int32), axis=1,
                                 keepdims=True)

        c_gt = lax.fori_loop(0, chunks, gt_chunk,
                             jnp.zeros((E, 1), jnp.int32))

        def ge_chunk(i, acc):
            return acc + jnp.sum((gm_ref[i] >= thr).astype(jnp.int32),
                                 axis=1, keepdims=True)

        n_ge = lax.fori_loop(0, chunks, ge_chunk,
                             jnp.zeros((E, 1), jnp.int32))
        thr_ref[...] = jnp.where(loads > cap, thr, 0)
        r_ref[...] = cap - jnp.where(loads > cap, c_gt, 0)
        tflag_ref[0] = jnp.any((loads > cap) & (n_ge > cap)).astype(jnp.int32)

    thr = thr_ref[...]
    r = r_ref[...]

    z = jnp.zeros((E, 1), jnp.int32)

    @pl.when(tflag_ref[0] == 1)
    def _scan_heavy():
        def scan_chunk(i, carry):
            kcnt, tcnt = carry
            g, gb, e, eoh = chunk(i)
            eoh_i = eoh.astype(jnp.int32)
            t_tok = jnp.sum(jnp.where(eoh, thr, 0), axis=0, keepdims=True)
            r_tok = jnp.sum(jnp.where(eoh, r, 0), axis=0, keepdims=True)
            gt = gb > t_tok
            tie = gb == t_tok
            tie_m = eoh_i * tie.astype(jnp.int32)
            tpre = _lane_cumsum(tie_m, ch) - tie_m
            trank = jnp.sum(jnp.where(eoh, tpre + tcnt, 0), axis=0,
                            keepdims=True)
            keep = gt | (tie & (trank < r_tok))
            keep_m = eoh_i * keep.astype(jnp.int32)
            kpre = _lane_cumsum(keep_m, ch) - keep_m
            slot = jnp.sum(jnp.where(eoh, kpre + kcnt, 0), axis=0,
                           keepdims=True)
            dest_ref[i] = jnp.where(keep, e * cap + slot, ecap)
            scale_ref[i] = jnp.where(keep, g, 0.0)
            kcnt = kcnt + jnp.sum(keep_m, axis=1, keepdims=True)
            tcnt = tcnt + jnp.sum(tie_m, axis=1, keepdims=True)
            return kcnt, tcnt

        lax.fori_loop(0, chunks, scan_chunk, (z, z))

    @pl.when(tflag_ref[0] == 0)
    def _scan_light():
        def scan_chunk(i, kcnt):
            g, gb, e, eoh = chunk(i)
            t_tok = jnp.sum(jnp.where(eoh, thr, 0), axis=0, keepdims=True)
            keep = gb >= t_tok
            keep_m = eoh.astype(jnp.int32) * keep.astype(jnp.int32)
            kpre = _lane_cumsum(keep_m, ch) - keep_m
            slot = jnp.sum(jnp.where(eoh, kpre + kcnt, 0), axis=0,
                           keepdims=True)
            dest_ref[i] = jnp.where(keep, e * cap + slot, ecap)
            scale_ref[i] = jnp.where(keep, g, 0.0)
            return kcnt + jnp.sum(keep_m, axis=1, keepdims=True)

        lax.fori_loop(0, chunks, scan_chunk, z)


def _ffn_body(buf_ref, w1_ref, b1_ref, w2_ref, b2_ref, o_ref):
    xb = buf_ref[...].astype(jnp.bfloat16)
    h = jnp.dot(xb, w1_ref[0].astype(jnp.bfloat16),
                preferred_element_type=jnp.float32)
    h = jax.nn.gelu(h + b1_ref[0])
    o = jnp.dot(h.astype(jnp.bfloat16), w2_ref[0].astype(jnp.bfloat16),
                preferred_element_type=jnp.float32)
    o_ref[...] = o + b2_ref[0]


def _out_body(ot_ref, scale_ref, wp_ref, bp_ref, y_ref):
    s = scale_ref[...][:, None]
    t = jnp.where(s > 0, ot_ref[...], 0.0) * s
    t = jax.nn.gelu(t)
    y_ref[...] = (jnp.dot(t, wp_ref[...], preferred_element_type=jnp.float32)
                  + bp_ref[...])


def _dispatch_body(n, xf_hbm, dest_hbm, buf_hbm, idx_v, rows_v, sem):
    wid = lax.axis_index("s") * 2 + lax.axis_index("c")
    per_w = n // 32
    base0 = wid * per_w

    def body(j, carry):
        base = base0 + j * 128
        pltpu.sync_copy(dest_hbm.at[pl.ds(base, 128)], idx_v)
        pltpu.sync_copy(xf_hbm.at[pl.ds(base, 128)], rows_v)
        pltpu.async_copy(rows_v, buf_hbm.at[idx_v], sem).wait()
        return carry

    lax.fori_loop(0, per_w // 128, body, 0)


def _combine_body(n, o_hbm, dest_hbm, out_hbm, idx_v, rows_v, sem):
    wid = lax.axis_index("s") * 2 + lax.axis_index("c")
    per_w = n // 32
    base0 = wid * per_w

    def body(j, carry):
        base = base0 + j * 128
        pltpu.sync_copy(dest_hbm.at[pl.ds(base, 128)], idx_v)
        pltpu.async_copy(o_hbm.at[idx_v], rows_v, sem).wait()
        pltpu.sync_copy(rows_v, out_hbm.at[pl.ds(base, 128)])
        return carry

    lax.fori_loop(0, per_w // 128, body, 0)


def kernel(x, embedding, Wg, W1, b1, W2, b2, Wp, bp):
    B, T, H, _ = x.shape
    N = B * T * H
    TH = T * H
    cap = int(math.ceil(CAPF * N / E))
    ecap = E * cap

    BLKA = 4096
    CH = 8192
    CHUNKS = N // CH
    BLKD = 3072
    RB = ecap + BLKD
    CPB = cap // BLKD
    BLKF = 8192

    x2 = x.reshape(N, D)
    emb2 = embedding.reshape(TH, D)

    xf, gate, eidx = pl.pallas_call(
        _gate_body,
        grid=(N // BLKA,),
        in_specs=[
            pl.BlockSpec((BLKA, D), lambda i: (i, 0)),
            pl.BlockSpec((BLKA, D), lambda i: (i % (TH // BLKA), 0)),
            pl.BlockSpec((D, E), lambda i: (0, 0)),
        ],
        out_specs=[
            pl.BlockSpec((BLKA, D), lambda i: (i, 0)),
            pl.BlockSpec((BLKA,), lambda i: (i,)),
            pl.BlockSpec((BLKA,), lambda i: (i,)),
        ],
        out_shape=[
            jax.ShapeDtypeStruct((N, D), jnp.float32),
            jax.ShapeDtypeStruct((N,), jnp.float32),
            jax.ShapeDtypeStruct((N,), jnp.int32),
        ],
    )(x2, emb2, Wg)

    g3 = gate.reshape(CHUNKS, 1, CH)
    e3 = eidx.reshape(CHUNKS, 1, CH)
    dest3, scale3 = pl.pallas_call(
        functools.partial(_routing_body, cap, CHUNKS, CH),
        out_shape=[
            jax.ShapeDtypeStruct((CHUNKS, 1, CH), jnp.int32),
            jax.ShapeDtypeStruct((CHUNKS, 1, CH), jnp.float32),
        ],
        scratch_shapes=[
            pltpu.VMEM((E, 1), jnp.int32),
            pltpu.VMEM((E, 1), jnp.int32),
            pltpu.VMEM((CHUNKS, E, CH), jnp.int32),
            pltpu.SMEM((1,), jnp.int32),
        ],
    )(g3, e3)
    dest = dest3.reshape(N)
    scale = scale3.reshape(N)

    mesh = plsc.VectorSubcoreMesh(core_axis_name="c", subcore_axis_name="s")
    buf = pl.kernel(
        functools.partial(_dispatch_body, N),
        mesh=mesh,
        out_type=jax.ShapeDtypeStruct((RB, D), jnp.float32),
        scratch_types=[
            pltpu.VMEM((128,), jnp.int32),
            pltpu.VMEM((128, D), jnp.float32),
            pltpu.SemaphoreType.DMA,
        ],
    )(xf, dest)

    o = pl.pallas_call(
        _ffn_body,
        grid=(RB // BLKD,),
        in_specs=[
            pl.BlockSpec((BLKD, D), lambda i: (i, 0)),
            pl.BlockSpec((1, D, FF),
                         lambda i: (jnp.minimum(i // CPB, E - 1), 0, 0)),
            pl.BlockSpec((1, 1, FF),
                         lambda i: (jnp.minimum(i // CPB, E - 1), 0, 0)),
            pl.BlockSpec((1, FF, D),
                         lambda i: (jnp.minimum(i // CPB, E - 1), 0, 0)),
            pl.BlockSpec((1, 1, D),
                         lambda i: (jnp.minimum(i // CPB, E - 1), 0, 0)),
        ],
        out_specs=pl.BlockSpec((BLKD, D), lambda i: (i, 0)),
        out_shape=jax.ShapeDtypeStruct((RB, D), jnp.float32),
    )(buf, W1, b1.reshape(E, 1, FF), W2, b2.reshape(E, 1, D))

    otok = pl.kernel(
        functools.partial(_combine_body, N),
        mesh=mesh,
        out_type=jax.ShapeDtypeStruct((N, D), jnp.float32),
        scratch_types=[
            pltpu.VMEM((128,), jnp.int32),
            pltpu.VMEM((128, D), jnp.float32),
            pltpu.SemaphoreType.DMA,
        ],
    )(o, dest)

    y = pl.pallas_call(
        _out_body,
        grid=(N // BLKF,),
        in_specs=[
            pl.BlockSpec((BLKF, D), lambda i: (i, 0)),
            pl.BlockSpec((BLKF,), lambda i: (i,)),
            pl.BlockSpec((D, OD), lambda i: (0, 0)),
            pl.BlockSpec((OD,), lambda i: (0,)),
        ],
        out_specs=pl.BlockSpec((BLKF, OD), lambda i: (i, 0)),
        out_shape=jax.ShapeDtypeStruct((N, OD), jnp.float32),
    )(otok, scale, Wp, bp)

    return y.reshape(B, T, H, OD)
